# baseline (speedup 1.0000x reference)
import functools

import jax
import jax.numpy as jnp
from jax.experimental import pallas as pl
from jax.experimental.pallas import tpu as pltpu


def _start_block(adj_hbm, abuf, asems, blk, buf, *, bm: int, nsplit: int):
    rows = bm // nsplit
    for s in range(nsplit):
        pltpu.make_async_copy(
            adj_hbm.at[pl.ds(blk * bm + s * rows, rows), :],
            abuf.at[buf, pl.ds(s * rows, rows), :],
            asems.at[buf, s],
        ).start()


def _wait_block(adj_hbm, abuf, asems, blk, buf, *, bm: int, nsplit: int):
    rows = bm // nsplit
    for s in range(nsplit):
        pltpu.make_async_copy(
            adj_hbm.at[pl.ds(blk * bm + s * rows, rows), :],
            abuf.at[buf, pl.ds(s * rows, rows), :],
            asems.at[buf, s],
        ).wait()


def _gcn_kernel(x_hbm, adj_hbm, w_ref, b_ref, out_ref,
                h_ref, xbuf, abuf, xsems, asems,
                *, m: int, chunk: int, bm: int, nsplit: int):
    i = pl.program_id(0)
    nblocks = pl.num_programs(0)

    @pl.when(i == 0)
    def _prologue():
        _start_block(adj_hbm, abuf, asems, 0, 0, bm=bm, nsplit=nsplit)
        _start_block(adj_hbm, abuf, asems, 1, 1, bm=bm, nsplit=nsplit)
        w = w_ref[...].astype(jnp.bfloat16)
        nchunks = m // chunk
        pltpu.make_async_copy(
            x_hbm.at[pl.ds(0, chunk), :], xbuf.at[0], xsems.at[0]).start()
        for c in range(nchunks):
            if c + 1 < nchunks:
                pltpu.make_async_copy(
                    x_hbm.at[pl.ds((c + 1) * chunk, chunk), :],
                    xbuf.at[(c + 1) % 2], xsems.at[(c + 1) % 2]).start()
            pltpu.make_async_copy(
                x_hbm.at[pl.ds(c * chunk, chunk), :],
                xbuf.at[c % 2], xsems.at[c % 2]).wait()
            h_ref[pl.ds(c * chunk, chunk), :] = jnp.dot(
                xbuf[c % 2].astype(jnp.bfloat16), w,
                preferred_element_type=jnp.float32,
            ).astype(jnp.bfloat16)

    @pl.when(jnp.logical_and(i > 0, i + 1 < nblocks))
    def _prefetch():
        _start_block(adj_hbm, abuf, asems, i + 1, (i + 1) % 2,
                     bm=bm, nsplit=nsplit)

    _wait_block(adj_hbm, abuf, asems, i, i % 2, bm=bm, nsplit=nsplit)
    a = abuf[i % 2].astype(jnp.bfloat16)
    acc = jnp.dot(a, h_ref[...], preferred_element_type=jnp.float32)
    out_ref[...] = acc + b_ref[...]


def kernel(input, adj, W, b):
    m, kin = input.shape
    kout = W.shape[1]
    n = adj.shape[1]

    bm = 400 if m % 400 == 0 else m
    chunk = 2000 if m % 2000 == 0 else m
    nsplit = 5 if bm % 40 == 0 else 1
    b2 = b.reshape(1, kout)

    body = functools.partial(_gcn_kernel, m=m, chunk=chunk, bm=bm,
                             nsplit=nsplit)
    out = pl.pallas_call(
        body,
        grid=(m // bm,),
        in_specs=[
            pl.BlockSpec(memory_space=pl.ANY),
            pl.BlockSpec(memory_space=pl.ANY),
            pl.BlockSpec((kin, kout), lambda i: (0, 0)),
            pl.BlockSpec((1, kout), lambda i: (0, 0)),
        ],
        out_specs=pl.BlockSpec((bm, kout), lambda i: (i, 0)),
        out_shape=jax.ShapeDtypeStruct((m, kout), jnp.float32),
        scratch_shapes=[
            pltpu.VMEM((m, kout), jnp.bfloat16),
            pltpu.VMEM((2, chunk, kin), jnp.float32),
            pltpu.VMEM((2, bm, n), jnp.float32),
            pltpu.SemaphoreType.DMA((2,)),
            pltpu.SemaphoreType.DMA((2, nsplit)),
        ],
        compiler_params=pltpu.CompilerParams(
            dimension_semantics=("arbitrary",),
            vmem_limit_bytes=64 * 1024 * 1024,
        ),
    )(input, adj, W, b2)
    return out


# speedup vs baseline: 1.0049x; 1.0049x over previous
import functools

import jax
import jax.numpy as jnp
from jax.experimental import pallas as pl
from jax.experimental.pallas import tpu as pltpu


def _gcn_kernel(x_hbm, w_ref, adj_ref, b_ref, out_ref, h_ref, xbuf, sems,
                *, m: int, chunk: int):
    i = pl.program_id(0)

    @pl.when(i == 0)
    def _compute_h():
        w = w_ref[...].astype(jnp.bfloat16)
        nchunks = m // chunk
        cp0 = pltpu.make_async_copy(
            x_hbm.at[pl.ds(0, chunk), :], xbuf.at[0], sems.at[0])
        cp0.start()
        for c in range(nchunks):
            if c + 1 < nchunks:
                cpn = pltpu.make_async_copy(
                    x_hbm.at[pl.ds((c + 1) * chunk, chunk), :],
                    xbuf.at[(c + 1) % 2], sems.at[(c + 1) % 2])
                cpn.start()
            pltpu.make_async_copy(
                x_hbm.at[pl.ds(c * chunk, chunk), :],
                xbuf.at[c % 2], sems.at[c % 2]).wait()
            h_ref[pl.ds(c * chunk, chunk), :] = jnp.dot(
                xbuf[c % 2].astype(jnp.bfloat16), w,
                preferred_element_type=jnp.float32,
            ).astype(jnp.bfloat16)

    a = adj_ref[...].astype(jnp.bfloat16)
    acc = jnp.dot(a, h_ref[...], preferred_element_type=jnp.float32)
    out_ref[...] = acc + b_ref[...]


def kernel(input, adj, W, b):
    m, kin = input.shape
    kout = W.shape[1]
    n = adj.shape[1]

    bm = 400 if m % 400 == 0 else m
    chunk = 2000 if m % 2000 == 0 else m
    b2 = b.reshape(1, kout)

    body = functools.partial(_gcn_kernel, m=m, chunk=chunk)
    out = pl.pallas_call(
        body,
        grid=(pl.cdiv(m, bm),),
        in_specs=[
            pl.BlockSpec(memory_space=pl.ANY),
            pl.BlockSpec((kin, kout), lambda i: (0, 0)),
            pl.BlockSpec((bm, n), lambda i: (i, 0)),
            pl.BlockSpec((1, kout), lambda i: (0, 0)),
        ],
        out_specs=pl.BlockSpec((bm, kout), lambda i: (i, 0)),
        out_shape=jax.ShapeDtypeStruct((m, kout), jnp.float32),
        scratch_shapes=[
            pltpu.VMEM((n, kout), jnp.bfloat16),
            pltpu.VMEM((2, chunk, kin), jnp.float32),
            pltpu.SemaphoreType.DMA((2,)),
        ],
        compiler_params=pltpu.CompilerParams(
            dimension_semantics=("arbitrary",),
            vmem_limit_bytes=64 * 1024 * 1024,
        ),
    )(input, W, adj, b2)
    return out
